# baseline (speedup 1.0000x reference)
import functools

import jax
import jax.numpy as jnp
import numpy as np
from jax.experimental import pallas as pl

_BASE = 2
_MAX_K = 10
_NCLS = _MAX_K + 1


def _onehot_body(x_ref, o_ref):
    ts = x_ref[...]
    shifted = jnp.concatenate([ts[:, :1], ts[:, :-1]], axis=1)
    itvl = ts - shifted
    lg = jnp.log(itvl) / jnp.log(jnp.float32(_BASE))
    lg = jnp.where(jnp.isnan(lg), jnp.zeros_like(lg), lg)
    bucket = jnp.clip(jnp.floor(lg), 0.0, float(_MAX_K)).astype(jnp.int32)
    cls = jax.lax.broadcasted_iota(jnp.int32, bucket.shape + (_NCLS,), 2)
    o_ref[...] = (bucket[:, :, None] == cls).astype(jnp.float32)


@jax.jit
def kernel(timestamps):
    B, T, _ = timestamps.shape
    x = timestamps.reshape(B, T)
    bb = 128
    out = pl.pallas_call(
        _onehot_body,
        grid=(B // bb,),
        in_specs=[pl.BlockSpec((bb, T), lambda i: (i, 0))],
        out_specs=pl.BlockSpec((bb, T, _NCLS), lambda i: (i, 0, 0)),
        out_shape=jax.ShapeDtypeStruct((B, T, _NCLS), jnp.float32),
    )(x)
    return out


# speedup vs baseline: 17.8807x; 17.8807x over previous
import jax
import jax.numpy as jnp
from jax.experimental import pallas as pl

_BASE = 2
_MAX_K = 10
_NCLS = _MAX_K + 1


def _onehot_body(x_ref, o_ref):
    ts = x_ref[...]
    shifted = jnp.concatenate([ts[:1], ts[:-1]], axis=0)
    itvl = ts - shifted
    lg = jnp.log(itvl) / jnp.log(jnp.float32(_BASE))
    lg = jnp.where(jnp.isnan(lg), jnp.zeros_like(lg), lg)
    bucket = jnp.clip(jnp.floor(lg), 0.0, float(_MAX_K)).astype(jnp.int32)
    for c in range(_NCLS):
        o_ref[c] = (bucket == c).astype(jnp.float32)


@jax.jit
def kernel(timestamps):
    B, T, _ = timestamps.shape
    xt = timestamps.reshape(B, T).T
    lb = 1024
    out = pl.pallas_call(
        _onehot_body,
        grid=(B // lb,),
        in_specs=[pl.BlockSpec((T, lb), lambda i: (0, i))],
        out_specs=pl.BlockSpec((_NCLS, T, lb), lambda i: (0, 0, i)),
        out_shape=jax.ShapeDtypeStruct((_NCLS, T, B), jnp.float32),
    )(xt)
    return jnp.transpose(out, (2, 1, 0))
